# baseline (speedup 1.0000x reference)
import jax
import jax.numpy as jnp
from jax import lax
from jax.experimental import pallas as pl
from jax.experimental.pallas import tpu as pltpu
from jax.experimental.pallas import tpu_sc as plsc

_B = 16384
_D = 64
_NW = 32
_BPW = _B // _NW
_ICH = 128
_NCH = _BPW // _ICH
_GROUPS = _BPW // 16

_NUP = 1024000
_NIP = 102400
_UBW = 102400
_IBW = 51200


def _matvec_body(w_ref, tab_ref, out_ref):
    out_ref[...] = jnp.dot(w_ref[...], tab_ref[...],
                           precision=lax.Precision.HIGHEST)[0]


def _row_scores(tab_t, w_half, n_pad, bw):
    nb = n_pad // bw
    return pl.pallas_call(
        _matvec_body,
        grid=(nb,),
        in_specs=[
            pl.BlockSpec((1, _D), lambda i: (0, 0)),
            pl.BlockSpec((_D, bw), lambda i: (0, i)),
        ],
        out_specs=pl.BlockSpec((bw,), lambda i: (i,)),
        out_shape=jax.ShapeDtypeStruct((n_pad,), jnp.float32),
    )(w_half, tab_t)


def _sc_body(idx_hbm, s_hbm, add_hbm, out_hbm,
             idx_v, g_v, rows_v, add_v, out_v, sem):
    wid = lax.axis_index("s") * 2 + lax.axis_index("c")
    base = wid * _BPW

    pltpu.sync_copy(idx_hbm.at[pl.ds(base, _BPW)], idx_v)
    pltpu.sync_copy(add_hbm.at[pl.ds(base, _BPW)], add_v)

    lane = lax.broadcasted_iota(jnp.int32, (16,), 0)

    def row_idx(c, carry):
        g_v[pl.ds(c * 16, 16)] = lax.shift_right_logical(
            idx_v[pl.ds(c * 16, 16)], 7)
        return carry
    lax.fori_loop(0, _GROUPS, row_idx, 0)

    copies = [
        pltpu.async_copy(s_hbm.at[g_v.at[pl.ds(k * _ICH, _ICH)]],
                         rows_v.at[pl.ds(k * _ICH, _ICH)], sem)
        for k in range(_NCH)
    ]
    for c in copies:
        c.wait()

    def group(g, carry):
        mvals = idx_v[pl.ds(g * 16, 16)] & 127
        acc = jnp.zeros((16,), jnp.float32)
        for rr in range(16):
            r = g * 16 + rr
            m = mvals.at[jnp.full((16,), rr, jnp.int32)].get(
                mode="promise_in_bounds")
            mlow = m & 15
            m1 = (lax.shift_right_logical(m, 4) & 1).astype(jnp.float32)
            m2 = (lax.shift_right_logical(m, 5) & 1).astype(jnp.float32)
            m3 = lax.shift_right_logical(m, 6).astype(jnp.float32)
            rv = [rows_v[r, pl.ds(c * 16, 16)] for c in range(8)]
            q = [rv[2 * i] + m1 * (rv[2 * i + 1] - rv[2 * i])
                 for i in range(4)]
            s = [q[2 * i] + m2 * (q[2 * i + 1] - q[2 * i])
                 for i in range(2)]
            t = s[0] + m3 * (s[1] - s[0])
            word = t.at[mlow].get(mode="promise_in_bounds")
            acc = jnp.where(lane == rr, word, acc)
        out_v[pl.ds(g * 16, 16)] = acc + add_v[pl.ds(g * 16, 16)]
        return carry
    lax.fori_loop(0, _GROUPS, group, 0)

    pltpu.sync_copy(out_v, out_hbm.at[pl.ds(base, _BPW)])


def _sc_gather_add(idx, score2d, addend):
    mesh = plsc.VectorSubcoreMesh(core_axis_name="c", subcore_axis_name="s")
    f = pl.kernel(
        _sc_body,
        out_type=jax.ShapeDtypeStruct((_B,), jnp.float32),
        mesh=mesh,
        compiler_params=pltpu.CompilerParams(use_tc_tiling_on_sc=True),
        scratch_types=[
            pltpu.VMEM((_BPW,), jnp.int32),
            pltpu.VMEM((_BPW,), jnp.int32),
            pltpu.VMEM((_BPW, 128), jnp.float32),
            pltpu.VMEM((_BPW,), jnp.float32),
            pltpu.VMEM((_BPW,), jnp.float32),
            pltpu.SemaphoreType.DMA,
        ],
    )
    return f(idx, score2d, addend)


def kernel(user_idx, item_idx, user_table, item_table, W, b):
    wu = W[:, :_D].astype(jnp.float32)
    wi = W[:, _D:].astype(jnp.float32)
    bb = jnp.broadcast_to(b.astype(jnp.float32), (_B,))

    iscore = _row_scores(item_table.T, wi, _NIP, _IBW)
    is2 = iscore.reshape(_NIP // 128, 128)
    partial = _sc_gather_add(item_idx.astype(jnp.int32), is2, bb)

    uscore = _row_scores(user_table.T, wu, _NUP, _UBW)
    us2 = uscore.reshape(_NUP // 128, 128)
    return _sc_gather_add(user_idx.astype(jnp.int32), us2, partial)


# speedup vs baseline: 1.0318x; 1.0318x over previous
import jax
import jax.numpy as jnp
from jax import lax
from jax.experimental import pallas as pl
from jax.experimental.pallas import tpu as pltpu
from jax.experimental.pallas import tpu_sc as plsc

_B = 16384
_D = 64
_NW = 32
_BPW = _B // _NW
_ICH = 128
_NCH = _BPW // _ICH
_GROUPS = _BPW // 16

_NUP = 1024000
_NIP = 102400
_UBW = 51200
_IBW = 25600


def _matvec_body(w_ref, tab_ref, out_ref):
    out_ref[...] = jnp.dot(w_ref[...], tab_ref[...],
                           precision=lax.Precision.HIGHEST)[0]


def _row_scores(tab_t, w_half, n_pad, bw):
    nb = n_pad // bw
    return pl.pallas_call(
        _matvec_body,
        grid=(nb,),
        in_specs=[
            pl.BlockSpec((1, _D), lambda i: (0, 0)),
            pl.BlockSpec((_D, bw), lambda i: (0, i)),
        ],
        out_specs=pl.BlockSpec((bw,), lambda i: (i,)),
        out_shape=jax.ShapeDtypeStruct((n_pad,), jnp.float32),
    )(w_half, tab_t)


def _sc_body(idx_hbm, s_hbm, add_hbm, out_hbm,
             idx_v, g_v, rows_v, add_v, out_v, sem):
    wid = lax.axis_index("s") * 2 + lax.axis_index("c")
    base = wid * _BPW

    pltpu.sync_copy(idx_hbm.at[pl.ds(base, _BPW)], idx_v)
    pltpu.sync_copy(add_hbm.at[pl.ds(base, _BPW)], add_v)

    lane = lax.broadcasted_iota(jnp.int32, (16,), 0)

    def row_idx(c, carry):
        g_v[pl.ds(c * 16, 16)] = lax.shift_right_logical(
            idx_v[pl.ds(c * 16, 16)], 7)
        return carry
    lax.fori_loop(0, _GROUPS, row_idx, 0)

    copies = [
        pltpu.async_copy(s_hbm.at[g_v.at[pl.ds(k * _ICH, _ICH)]],
                         rows_v.at[pl.ds(k * _ICH, _ICH)], sem)
        for k in range(_NCH)
    ]
    for c in copies:
        c.wait()

    def group(g, carry):
        mvals = idx_v[pl.ds(g * 16, 16)] & 127
        acc = jnp.zeros((16,), jnp.float32)
        for rr in range(16):
            r = g * 16 + rr
            m = mvals.at[jnp.full((16,), rr, jnp.int32)].get(
                mode="promise_in_bounds")
            mlow = m & 15
            m1 = (lax.shift_right_logical(m, 4) & 1).astype(jnp.float32)
            m2 = (lax.shift_right_logical(m, 5) & 1).astype(jnp.float32)
            m3 = lax.shift_right_logical(m, 6).astype(jnp.float32)
            rv = [rows_v[r, pl.ds(c * 16, 16)] for c in range(8)]
            q = [rv[2 * i] + m1 * (rv[2 * i + 1] - rv[2 * i])
                 for i in range(4)]
            s = [q[2 * i] + m2 * (q[2 * i + 1] - q[2 * i])
                 for i in range(2)]
            t = s[0] + m3 * (s[1] - s[0])
            word = t.at[mlow].get(mode="promise_in_bounds")
            acc = jnp.where(lane == rr, word, acc)
        out_v[pl.ds(g * 16, 16)] = acc + add_v[pl.ds(g * 16, 16)]
        return carry
    lax.fori_loop(0, _GROUPS, group, 0)

    pltpu.sync_copy(out_v, out_hbm.at[pl.ds(base, _BPW)])


def _sc_gather_add(idx, score2d, addend):
    mesh = plsc.VectorSubcoreMesh(core_axis_name="c", subcore_axis_name="s")
    f = pl.kernel(
        _sc_body,
        out_type=jax.ShapeDtypeStruct((_B,), jnp.float32),
        mesh=mesh,
        compiler_params=pltpu.CompilerParams(use_tc_tiling_on_sc=True),
        scratch_types=[
            pltpu.VMEM((_BPW,), jnp.int32),
            pltpu.VMEM((_BPW,), jnp.int32),
            pltpu.VMEM((_BPW, 128), jnp.float32),
            pltpu.VMEM((_BPW,), jnp.float32),
            pltpu.VMEM((_BPW,), jnp.float32),
            pltpu.SemaphoreType.DMA,
        ],
    )
    return f(idx, score2d, addend)


def kernel(user_idx, item_idx, user_table, item_table, W, b):
    wu = W[:, :_D].astype(jnp.float32)
    wi = W[:, _D:].astype(jnp.float32)
    bb = jnp.broadcast_to(b.astype(jnp.float32), (_B,))

    iscore = _row_scores(item_table.T, wi, _NIP, _IBW)
    is2 = iscore.reshape(_NIP // 128, 128)
    partial = _sc_gather_add(item_idx.astype(jnp.int32), is2, bb)

    uscore = _row_scores(user_table.T, wu, _NUP, _UBW)
    us2 = uscore.reshape(_NUP // 128, 128)
    return _sc_gather_add(user_idx.astype(jnp.int32), us2, partial)


# speedup vs baseline: 1.0318x; 1.0000x over previous
import functools

import jax
import jax.numpy as jnp
from jax import lax
from jax.experimental import pallas as pl
from jax.experimental.pallas import tpu as pltpu
from jax.experimental.pallas import tpu_sc as plsc

_B = 16384
_D = 64
_NW = 32
_BPW = _B // _NW
_ICH = 128
_NCH = _BPW // _ICH
_GROUPS = _BPW // 16

_NUP = 1024000
_NIP = 102400
_UBW = 51200
_IBW = 25600


def _matvec_body(w_ref, tab_ref, out_ref, *, n_valid, bw):
    res = jnp.dot(w_ref[...], tab_ref[...],
                  precision=lax.Precision.HIGHEST)
    col = (lax.broadcasted_iota(jnp.int32, res.shape, 1)
           + pl.program_id(0) * bw)
    out_ref[...] = jnp.where(col < n_valid, res, 0.0)[0]


def _row_scores(tab_t, w_half, n_pad, bw):
    nb = n_pad // bw
    return pl.pallas_call(
        functools.partial(_matvec_body, n_valid=tab_t.shape[1], bw=bw),
        grid=(nb,),
        in_specs=[
            pl.BlockSpec((1, _D), lambda i: (0, 0)),
            pl.BlockSpec((_D, bw), lambda i: (0, i)),
        ],
        out_specs=pl.BlockSpec((bw,), lambda i: (i,)),
        out_shape=jax.ShapeDtypeStruct((n_pad,), jnp.float32),
    )(w_half, tab_t)


def _sc_body(idx_hbm, s_hbm, add_hbm, out_hbm,
             idx_v, g_v, rows_v, add_v, out_v, sem):
    wid = lax.axis_index("s") * 2 + lax.axis_index("c")
    base = wid * _BPW

    pltpu.sync_copy(idx_hbm.at[pl.ds(base, _BPW)], idx_v)
    pltpu.sync_copy(add_hbm.at[pl.ds(base, _BPW)], add_v)

    lane = lax.broadcasted_iota(jnp.int32, (16,), 0)

    def row_idx(c, carry):
        g_v[pl.ds(c * 16, 16)] = lax.shift_right_logical(
            idx_v[pl.ds(c * 16, 16)], 7)
        return carry
    lax.fori_loop(0, _GROUPS, row_idx, 0)

    copies = [
        pltpu.async_copy(s_hbm.at[g_v.at[pl.ds(k * _ICH, _ICH)]],
                         rows_v.at[pl.ds(k * _ICH, _ICH)], sem)
        for k in range(_NCH)
    ]
    for c in copies:
        c.wait()

    def group(g, carry):
        mvals = idx_v[pl.ds(g * 16, 16)] & 127
        acc = jnp.zeros((16,), jnp.float32)
        for rr in range(16):
            r = g * 16 + rr
            m = mvals.at[jnp.full((16,), rr, jnp.int32)].get(
                mode="promise_in_bounds")
            mlow = m & 15
            m1 = (lax.shift_right_logical(m, 4) & 1).astype(jnp.float32)
            m2 = (lax.shift_right_logical(m, 5) & 1).astype(jnp.float32)
            m3 = lax.shift_right_logical(m, 6).astype(jnp.float32)
            rv = [rows_v[r, pl.ds(c * 16, 16)] for c in range(8)]
            q = [rv[2 * i] + m1 * (rv[2 * i + 1] - rv[2 * i])
                 for i in range(4)]
            s = [q[2 * i] + m2 * (q[2 * i + 1] - q[2 * i])
                 for i in range(2)]
            t = s[0] + m3 * (s[1] - s[0])
            word = t.at[mlow].get(mode="promise_in_bounds")
            acc = jnp.where(lane == rr, word, acc)
        out_v[pl.ds(g * 16, 16)] = acc + add_v[pl.ds(g * 16, 16)]
        return carry
    lax.fori_loop(0, _GROUPS, group, 0)

    pltpu.sync_copy(out_v, out_hbm.at[pl.ds(base, _BPW)])


def _sc_gather_add(idx, score2d, addend):
    mesh = plsc.VectorSubcoreMesh(core_axis_name="c", subcore_axis_name="s")
    f = pl.kernel(
        _sc_body,
        out_type=jax.ShapeDtypeStruct((_B,), jnp.float32),
        mesh=mesh,
        compiler_params=pltpu.CompilerParams(use_tc_tiling_on_sc=True),
        scratch_types=[
            pltpu.VMEM((_BPW,), jnp.int32),
            pltpu.VMEM((_BPW,), jnp.int32),
            pltpu.VMEM((_BPW, 128), jnp.float32),
            pltpu.VMEM((_BPW,), jnp.float32),
            pltpu.VMEM((_BPW,), jnp.float32),
            pltpu.SemaphoreType.DMA,
        ],
    )
    return f(idx, score2d, addend)


def kernel(user_idx, item_idx, user_table, item_table, W, b):
    wu = W[:, :_D].astype(jnp.float32)
    wi = W[:, _D:].astype(jnp.float32)
    bb = jnp.broadcast_to(b.astype(jnp.float32), (_B,))

    iscore = _row_scores(item_table.T, wi, _NIP, _IBW)
    is2 = iscore.reshape(_NIP // 128, 128)
    partial = _sc_gather_add(item_idx.astype(jnp.int32), is2, bb)

    uscore = _row_scores(user_table.T, wu, _NUP, _UBW)
    us2 = uscore.reshape(_NUP // 128, 128)
    return _sc_gather_add(user_idx.astype(jnp.int32), us2, partial)
